# baseline (speedup 1.0000x reference)
import numpy as np
import jax
import jax.numpy as jnp
from jax.experimental import pallas as pl

_NS = 8
_NPAM = 4
_K = 64
_CLIP = 20.0
_BIG = 1e9
_ISQ10 = np.float32(1.0 / np.sqrt(10.0))
_CONST = (np.array([-3.0, -1.0, 1.0, 3.0], dtype=np.float32) * _ISQ10)


def _ce(d_arr, p_arr, dist, asc_mask=None):
    N, B = d_arr.shape
    nb = N // (2 * dist)
    d = d_arr.reshape(nb, 2, dist, B)
    p = p_arr.reshape(nb, 2, dist, B)
    a, b = d[:, 0], d[:, 1]
    pa, pb = p[:, 0], p[:, 1]
    le = a <= b
    if asc_mask is None:
        le_eff = le
    else:
        le_eff = le == asc_mask
    first = jnp.where(le_eff, a, b)
    second = jnp.where(le_eff, b, a)
    pfirst = jnp.where(le_eff, pa, pb)
    psecond = jnp.where(le_eff, pb, pa)
    d_out = jnp.stack([first, second], axis=1).reshape(N, B)
    p_out = jnp.stack([pfirst, psecond], axis=1).reshape(N, B)
    return d_out, p_out


def _blk_iota(nb, dist):
    return jax.lax.broadcasted_iota(jnp.int32, (nb, 1, 1), 0) * (2 * dist)


def _select64(d_arr, p_arr):
    N = 256
    for k in range(2, 7):
        d = 2 ** (k - 1)
        while d >= 1:
            nb = N // (2 * d)
            asc = ((_blk_iota(nb, d) >> k) & 1) == 0
            d_arr, p_arr = _ce(d_arr, p_arr, d, asc)
            d //= 2
    d_arr, p_arr = _ce(d_arr, p_arr, 64)
    d2 = jnp.concatenate([d_arr[0:64], d_arr[128:192]], axis=0)
    p2 = jnp.concatenate([p_arr[0:64], p_arr[128:192]], axis=0)
    for d in (32, 16, 8, 4, 2, 1):
        nb = 128 // (2 * d)
        asc = _blk_iota(nb, d) < 64
        d2, p2 = _ce(d2, p2, d, asc)
    d2, p2 = _ce(d2, p2, 64)
    return d2[0:64], p2[0:64]


def _kbest_block(h_ref, y_ref, s_ref, out_ref):
    h = h_ref[...]
    y = y_ref[...]
    s = s_ref[...]
    Bb = y.shape[-1]
    sinv = 1.0 / s
    hd = h * sinv[:, None, :]
    G = jnp.zeros((8, 8, Bb), jnp.float32)
    z = jnp.zeros((8, Bb), jnp.float32)
    for t in range(16):
        G = G + hd[t][:, None, :] * h[t][None, :, :]
        z = z + hd[t] * y[t][None, :]
    n = jnp.stack([G[i, i] for i in range(8)], axis=0)
    jlt = (jax.lax.broadcasted_iota(jnp.int32, (8, 8, 1), 1)
           < jax.lax.broadcasted_iota(jnp.int32, (8, 8, 1), 0))
    gt = n[None, :, :] > n[:, None, :]
    eq = (n[None, :, :] == n[:, None, :]) & jlt
    rank = jnp.sum((gt | eq).astype(jnp.int32), axis=1)
    P = (rank[None, :, :] == jnp.arange(8, dtype=jnp.int32)[:, None, None]
         ).astype(jnp.float32)
    tmp = jnp.zeros((8, 8, Bb), jnp.float32)
    for i in range(8):
        tmp = tmp + P[:, i][:, None, :] * G[i][None, :, :]
    Gp = jnp.zeros((8, 8, Bb), jnp.float32)
    for j in range(8):
        Gp = Gp + tmp[:, j][:, None, :] * P[:, j][None, :, :]
    zp = jnp.zeros((8, Bb), jnp.float32)
    for i in range(8):
        zp = zp + P[:, i] * z[i][None, :]
    L = [[None] * 8 for _ in range(8)]
    D = [None] * 8
    for j in range(8):
        acc = Gp[j, j]
        for k in range(j):
            acc = acc - L[j][k] * L[j][k] * D[k]
        D[j] = acc
        for i in range(j + 1, 8):
            a2 = Gp[i, j]
            for k in range(j):
                a2 = a2 - L[i][k] * L[j][k] * D[k]
            L[i][j] = a2 / D[j]
    u = [None] * 8
    for i in range(8):
        acc = zp[i]
        for k in range(i):
            acc = acc - L[i][k] * u[k]
        u[i] = acc
    v = [u[i] / D[i] for i in range(8)]
    dists = jnp.zeros((1, Bb), jnp.float32)
    packed = jnp.zeros((1, Bb), jnp.int32)
    for stream in range(_NS):
        si = _NS - 1 - stream
        Pcur = dists.shape[0]
        m = jnp.zeros((Pcur, Bb), jnp.float32)
        for j in range(si + 1, 8):
            ind = (packed >> (2 * j)) & 3
            sym = (2.0 * ind.astype(jnp.float32) - 3.0) * _ISQ10
            m = m + L[j][si][None, :] * sym
        resid = v[si][None, :] - m
        dd = D[si][None, :]
        newd, newp = [], []
        for c in range(_NPAM):
            t = resid - _CONST[c]
            newd.append(dists + dd * t * t)
            newp.append(packed | np.int32(c << (2 * si)))
        if 4 * Pcur <= _K:
            dists = jnp.concatenate(newd, axis=0)
            packed = jnp.concatenate(newp, axis=0)
        else:
            d_e = jnp.stack(newd, axis=1).reshape(4 * Pcur, Bb)
            p_e = jnp.stack(newp, axis=1).reshape(4 * Pcur, Bb)
            dists, packed = _select64(d_e, p_e)
    ind = []
    for j in range(8):
        ind.append((packed >> (2 * rank[j][None, :])) & 3)
    for i in range(4):
        qam = ind[i] * _NPAM + ind[i + 4]
        for bit in range(4):
            b = (qam >> (3 - bit)) & 1
            d0 = jnp.min(jnp.where(b == 0, dists, _BIG), axis=0)
            d1 = jnp.min(jnp.where(b == 1, dists, _BIG), axis=0)
            out_ref[i * 4 + bit, :] = jnp.clip(d0 - d1, -_CLIP, _CLIP)


def kernel(y, h, s_diag):
    B = y.shape[0]
    Bblk = 256
    ht = jnp.transpose(h, (1, 2, 0))
    yt = jnp.transpose(y, (1, 0))
    st = jnp.transpose(s_diag, (1, 0))
    out = pl.pallas_call(
        _kbest_block,
        grid=(B // Bblk,),
        in_specs=[
            pl.BlockSpec((16, 8, Bblk), lambda i: (0, 0, i)),
            pl.BlockSpec((16, Bblk), lambda i: (0, i)),
            pl.BlockSpec((16, Bblk), lambda i: (0, i)),
        ],
        out_specs=pl.BlockSpec((16, Bblk), lambda i: (0, i)),
        out_shape=jax.ShapeDtypeStruct((16, B), jnp.float32),
    )(ht, yt, st)
    return jnp.transpose(out, (1, 0)).reshape(B, 4, 4)


# speedup vs baseline: 1.8022x; 1.8022x over previous
import numpy as np
import jax
import jax.numpy as jnp
from jax.experimental import pallas as pl

_NS = 8
_NPAM = 4
_K = 64
_CLIP = 20.0
_BIG = 1e9
_ISQ10 = np.float32(1.0 / np.sqrt(10.0))
_CONST = (np.array([-3.0, -1.0, 1.0, 3.0], dtype=np.float32) * _ISQ10)


def _ce(d_arr, p_arr, dist, asc_mask=None):
    N, B = d_arr.shape
    nb = N // (2 * dist)
    d = d_arr.reshape(nb, 2, dist, B)
    p = p_arr.reshape(nb, 2, dist, B)
    a, b = d[:, 0], d[:, 1]
    pa, pb = p[:, 0], p[:, 1]
    le = a <= b
    if asc_mask is None:
        le_eff = le
    else:
        le_eff = le == asc_mask
    first = jnp.where(le_eff, a, b)
    second = jnp.where(le_eff, b, a)
    pfirst = jnp.where(le_eff, pa, pb)
    psecond = jnp.where(le_eff, pb, pa)
    d_out = jnp.stack([first, second], axis=1).reshape(N, B)
    p_out = jnp.stack([pfirst, psecond], axis=1).reshape(N, B)
    return d_out, p_out


def _blk_iota(nb, dist):
    return jax.lax.broadcasted_iota(jnp.int32, (nb, 1, 1), 0) * (2 * dist)


def _select64(d_arr, p_arr):
    N = 256
    for k in range(2, 7):
        d = 2 ** (k - 1)
        while d >= 1:
            nb = N // (2 * d)
            asc = ((_blk_iota(nb, d) >> k) & 1) == 0
            d_arr, p_arr = _ce(d_arr, p_arr, d, asc)
            d //= 2
    d_arr, p_arr = _ce(d_arr, p_arr, 64)
    d2 = jnp.concatenate([d_arr[0:64], d_arr[128:192]], axis=0)
    p2 = jnp.concatenate([p_arr[0:64], p_arr[128:192]], axis=0)
    for d in (32, 16, 8, 4, 2, 1):
        nb = 128 // (2 * d)
        asc = _blk_iota(nb, d) < 64
        d2, p2 = _ce(d2, p2, d, asc)
    d2, p2 = _ce(d2, p2, 64)
    return d2[0:64], p2[0:64]


def _kbest_block(h_ref, y_ref, s_ref, out_ref):
    h = h_ref[...]
    y = y_ref[...]
    s = s_ref[...]
    Bb = y.shape[-1]
    sinv = 1.0 / s
    hd = h * sinv[:, None, :]
    G = jnp.zeros((8, 8, Bb), jnp.float32)
    z = jnp.zeros((8, Bb), jnp.float32)
    for t in range(16):
        G = G + hd[t][:, None, :] * h[t][None, :, :]
        z = z + hd[t] * y[t][None, :]
    n = jnp.stack([G[i, i] for i in range(8)], axis=0)
    jlt = (jax.lax.broadcasted_iota(jnp.int32, (8, 8, 1), 1)
           < jax.lax.broadcasted_iota(jnp.int32, (8, 8, 1), 0))
    gt = n[None, :, :] > n[:, None, :]
    eq = (n[None, :, :] == n[:, None, :]) & jlt
    rank = jnp.sum((gt | eq).astype(jnp.int32), axis=1)
    P = (rank[None, :, :] == jnp.arange(8, dtype=jnp.int32)[:, None, None]
         ).astype(jnp.float32)
    tmp = jnp.zeros((8, 8, Bb), jnp.float32)
    for i in range(8):
        tmp = tmp + P[:, i][:, None, :] * G[i][None, :, :]
    Gp = jnp.zeros((8, 8, Bb), jnp.float32)
    for j in range(8):
        Gp = Gp + tmp[:, j][:, None, :] * P[:, j][None, :, :]
    zp = jnp.zeros((8, Bb), jnp.float32)
    for i in range(8):
        zp = zp + P[:, i] * z[i][None, :]
    L = [[None] * 8 for _ in range(8)]
    D = [None] * 8
    for j in range(8):
        acc = Gp[j, j]
        for k in range(j):
            acc = acc - L[j][k] * L[j][k] * D[k]
        D[j] = acc
        for i in range(j + 1, 8):
            a2 = Gp[i, j]
            for k in range(j):
                a2 = a2 - L[i][k] * L[j][k] * D[k]
            L[i][j] = a2 / D[j]
    u = [None] * 8
    for i in range(8):
        acc = zp[i]
        for k in range(i):
            acc = acc - L[i][k] * u[k]
        u[i] = acc
    v = [u[i] / D[i] for i in range(8)]
    dists = jnp.zeros((1, Bb), jnp.float32)
    packed = jnp.zeros((1, Bb), jnp.int32)
    for stream in range(_NS):
        si = _NS - 1 - stream
        Pcur = dists.shape[0]
        m = jnp.zeros((Pcur, Bb), jnp.float32)
        for j in range(si + 1, 8):
            ind = (packed >> (2 * j)) & 3
            sym = (2.0 * ind.astype(jnp.float32) - 3.0) * _ISQ10
            m = m + L[j][si][None, :] * sym
        resid = v[si][None, :] - m
        dd = D[si][None, :]
        newd, newp = [], []
        for c in range(_NPAM):
            t = resid - _CONST[c]
            newd.append(dists + dd * t * t)
            newp.append(packed | np.int32(c << (2 * si)))
        if 4 * Pcur <= _K:
            dists = jnp.concatenate(newd, axis=0)
            packed = jnp.concatenate(newp, axis=0)
        else:
            d_e = jnp.stack(newd, axis=1).reshape(4 * Pcur, Bb)
            p_e = jnp.stack(newp, axis=1).reshape(4 * Pcur, Bb)
            dists, packed = _select64(d_e, p_e)
    ind = []
    for j in range(8):
        ind.append((packed >> (2 * rank[j][None, :])) & 3)
    for i in range(4):
        qam = ind[i] * _NPAM + ind[i + 4]
        for bit in range(4):
            b = (qam >> (3 - bit)) & 1
            d0 = jnp.min(jnp.where(b == 0, dists, _BIG), axis=0)
            d1 = jnp.min(jnp.where(b == 1, dists, _BIG), axis=0)
            out_ref[i * 4 + bit, :] = jnp.clip(d0 - d1, -_CLIP, _CLIP)


def kernel(y, h, s_diag):
    B = y.shape[0]
    Bblk = 512
    ht = jnp.transpose(h, (1, 2, 0))
    yt = jnp.transpose(y, (1, 0))
    st = jnp.transpose(s_diag, (1, 0))
    out = pl.pallas_call(
        _kbest_block,
        grid=(B // Bblk,),
        in_specs=[
            pl.BlockSpec((16, 8, Bblk), lambda i: (0, 0, i)),
            pl.BlockSpec((16, Bblk), lambda i: (0, i)),
            pl.BlockSpec((16, Bblk), lambda i: (0, i)),
        ],
        out_specs=pl.BlockSpec((16, Bblk), lambda i: (0, i)),
        out_shape=jax.ShapeDtypeStruct((16, B), jnp.float32),
    )(ht, yt, st)
    return jnp.transpose(out, (1, 0)).reshape(B, 4, 4)


# speedup vs baseline: 2.8496x; 1.5812x over previous
import numpy as np
import jax
import jax.numpy as jnp
from jax.experimental import pallas as pl

_NS = 8
_NPAM = 4
_K = 64
_CLIP = 20.0
_BIG = 1e9
_ISQ10 = np.float32(1.0 / np.sqrt(10.0))
_CONST = (np.array([-3.0, -1.0, 1.0, 3.0], dtype=np.float32) * _ISQ10)


def _ce(d_arr, p_arr, dist, asc_mask=None):
    N, B = d_arr.shape
    nb = N // (2 * dist)
    d = d_arr.reshape(nb, 2, dist, B)
    p = p_arr.reshape(nb, 2, dist, B)
    a, b = d[:, 0], d[:, 1]
    pa, pb = p[:, 0], p[:, 1]
    le = a <= b
    if asc_mask is None:
        le_eff = le
    else:
        le_eff = le == asc_mask
    first = jnp.where(le_eff, a, b)
    second = jnp.where(le_eff, b, a)
    pfirst = jnp.where(le_eff, pa, pb)
    psecond = jnp.where(le_eff, pb, pa)
    d_out = jnp.stack([first, second], axis=1).reshape(N, B)
    p_out = jnp.stack([pfirst, psecond], axis=1).reshape(N, B)
    return d_out, p_out


def _blk_iota(nb, dist):
    return jax.lax.broadcasted_iota(jnp.int32, (nb, 1, 1), 0) * (2 * dist)


def _select64(d_arr, p_arr):
    N = 256
    for k in range(2, 7):
        d = 2 ** (k - 1)
        while d >= 1:
            nb = N // (2 * d)
            asc = ((_blk_iota(nb, d) >> k) & 1) == 0
            d_arr, p_arr = _ce(d_arr, p_arr, d, asc)
            d //= 2
    d_arr, p_arr = _ce(d_arr, p_arr, 64)
    d2 = jnp.concatenate([d_arr[0:64], d_arr[128:192]], axis=0)
    p2 = jnp.concatenate([p_arr[0:64], p_arr[128:192]], axis=0)
    for d in (32, 16, 8, 4, 2, 1):
        nb = 128 // (2 * d)
        asc = _blk_iota(nb, d) < 64
        d2, p2 = _ce(d2, p2, d, asc)
    d2, p2 = _ce(d2, p2, 64)
    return d2[0:64], p2[0:64]


def _kbest_block(h_ref, y_ref, s_ref, out_ref):
    h = h_ref[...]
    y = y_ref[...]
    s = s_ref[...]
    Bb = y.shape[-1]
    sinv = 1.0 / s
    hd = h * sinv[:, None, :]
    G = jnp.zeros((8, 8, Bb), jnp.float32)
    z = jnp.zeros((8, Bb), jnp.float32)
    for t in range(16):
        G = G + hd[t][:, None, :] * h[t][None, :, :]
        z = z + hd[t] * y[t][None, :]
    n = jnp.stack([G[i, i] for i in range(8)], axis=0)
    jlt = (jax.lax.broadcasted_iota(jnp.int32, (8, 8, 1), 1)
           < jax.lax.broadcasted_iota(jnp.int32, (8, 8, 1), 0))
    gt = n[None, :, :] > n[:, None, :]
    eq = (n[None, :, :] == n[:, None, :]) & jlt
    rank = jnp.sum((gt | eq).astype(jnp.int32), axis=1)
    P = (rank[None, :, :] == jnp.arange(8, dtype=jnp.int32)[:, None, None]
         ).astype(jnp.float32)
    tmp = jnp.zeros((8, 8, Bb), jnp.float32)
    for i in range(8):
        tmp = tmp + P[:, i][:, None, :] * G[i][None, :, :]
    Gp = jnp.zeros((8, 8, Bb), jnp.float32)
    for j in range(8):
        Gp = Gp + tmp[:, j][:, None, :] * P[:, j][None, :, :]
    zp = jnp.zeros((8, Bb), jnp.float32)
    for i in range(8):
        zp = zp + P[:, i] * z[i][None, :]
    L = [[None] * 8 for _ in range(8)]
    D = [None] * 8
    for j in range(8):
        acc = Gp[j, j]
        for k in range(j):
            acc = acc - L[j][k] * L[j][k] * D[k]
        D[j] = acc
        for i in range(j + 1, 8):
            a2 = Gp[i, j]
            for k in range(j):
                a2 = a2 - L[i][k] * L[j][k] * D[k]
            L[i][j] = a2 / D[j]
    u = [None] * 8
    for i in range(8):
        acc = zp[i]
        for k in range(i):
            acc = acc - L[i][k] * u[k]
        u[i] = acc
    v = [u[i] / D[i] for i in range(8)]
    dists = jnp.zeros((1, Bb), jnp.float32)
    packed = jnp.zeros((1, Bb), jnp.int32)
    for stream in range(_NS):
        si = _NS - 1 - stream
        Pcur = dists.shape[0]
        m = jnp.zeros((Pcur, Bb), jnp.float32)
        for j in range(si + 1, 8):
            ind = (packed >> (2 * j)) & 3
            sym = (2.0 * ind.astype(jnp.float32) - 3.0) * _ISQ10
            m = m + L[j][si][None, :] * sym
        resid = v[si][None, :] - m
        dd = D[si][None, :]
        newd, newp = [], []
        for c in range(_NPAM):
            t = resid - _CONST[c]
            newd.append(dists + dd * t * t)
            newp.append(packed | np.int32(c << (2 * si)))
        if 4 * Pcur <= _K:
            dists = jnp.concatenate(newd, axis=0)
            packed = jnp.concatenate(newp, axis=0)
        else:
            d_e = jnp.stack(newd, axis=1).reshape(4 * Pcur, Bb)
            p_e = jnp.stack(newp, axis=1).reshape(4 * Pcur, Bb)
            dists, packed = _select64(d_e, p_e)
    ind = []
    for j in range(8):
        ind.append((packed >> (2 * rank[j][None, :])) & 3)
    for i in range(4):
        qam = ind[i] * _NPAM + ind[i + 4]
        for bit in range(4):
            b = (qam >> (3 - bit)) & 1
            d0 = jnp.min(jnp.where(b == 0, dists, _BIG), axis=0)
            d1 = jnp.min(jnp.where(b == 1, dists, _BIG), axis=0)
            out_ref[i * 4 + bit, :] = jnp.clip(d0 - d1, -_CLIP, _CLIP)


def kernel(y, h, s_diag):
    B = y.shape[0]
    Bblk = 1024
    ht = jnp.transpose(h, (1, 2, 0))
    yt = jnp.transpose(y, (1, 0))
    st = jnp.transpose(s_diag, (1, 0))
    out = pl.pallas_call(
        _kbest_block,
        grid=(B // Bblk,),
        in_specs=[
            pl.BlockSpec((16, 8, Bblk), lambda i: (0, 0, i)),
            pl.BlockSpec((16, Bblk), lambda i: (0, i)),
            pl.BlockSpec((16, Bblk), lambda i: (0, i)),
        ],
        out_specs=pl.BlockSpec((16, Bblk), lambda i: (0, i)),
        out_shape=jax.ShapeDtypeStruct((16, B), jnp.float32),
    )(ht, yt, st)
    return jnp.transpose(out, (1, 0)).reshape(B, 4, 4)


# speedup vs baseline: 2.9216x; 1.0253x over previous
import numpy as np
import jax
import jax.numpy as jnp
from jax.experimental import pallas as pl

_NS = 8
_NPAM = 4
_K = 64
_CLIP = 20.0
_BIG = 1e9
_ISQ10 = np.float32(1.0 / np.sqrt(10.0))
_CONST = (np.array([-3.0, -1.0, 1.0, 3.0], dtype=np.float32) * _ISQ10)


def _ce(d_arr, p_arr, dist, asc_mask=None):
    N, B = d_arr.shape
    nb = N // (2 * dist)
    d = d_arr.reshape(nb, 2, dist, B)
    p = p_arr.reshape(nb, 2, dist, B)
    a, b = d[:, 0], d[:, 1]
    pa, pb = p[:, 0], p[:, 1]
    le = a <= b
    if asc_mask is None:
        le_eff = le
    else:
        le_eff = le == asc_mask
    first = jnp.where(le_eff, a, b)
    second = jnp.where(le_eff, b, a)
    pfirst = jnp.where(le_eff, pa, pb)
    psecond = jnp.where(le_eff, pb, pa)
    d_out = jnp.stack([first, second], axis=1).reshape(N, B)
    p_out = jnp.stack([pfirst, psecond], axis=1).reshape(N, B)
    return d_out, p_out


def _blk_iota(nb, dist):
    return jax.lax.broadcasted_iota(jnp.int32, (nb, 1, 1), 0) * (2 * dist)


def _select64(d_arr, p_arr):
    N = 256
    for k in range(2, 7):
        d = 2 ** (k - 1)
        while d >= 1:
            nb = N // (2 * d)
            asc = ((_blk_iota(nb, d) >> k) & 1) == 0
            d_arr, p_arr = _ce(d_arr, p_arr, d, asc)
            d //= 2
    d_arr, p_arr = _ce(d_arr, p_arr, 64)
    d2 = jnp.concatenate([d_arr[0:64], d_arr[128:192]], axis=0)
    p2 = jnp.concatenate([p_arr[0:64], p_arr[128:192]], axis=0)
    for d in (32, 16, 8, 4, 2, 1):
        nb = 128 // (2 * d)
        asc = _blk_iota(nb, d) < 64
        d2, p2 = _ce(d2, p2, d, asc)
    d2, p2 = _ce(d2, p2, 64)
    return d2[0:64], p2[0:64]


def _kbest_block(h_ref, y_ref, s_ref, out_ref):
    h = h_ref[...]
    y = y_ref[...]
    s = s_ref[...]
    Bb = y.shape[-1]
    sinv = 1.0 / s
    hd = h * sinv[:, None, :]
    G = jnp.zeros((8, 8, Bb), jnp.float32)
    z = jnp.zeros((8, Bb), jnp.float32)
    for t in range(16):
        G = G + hd[t][:, None, :] * h[t][None, :, :]
        z = z + hd[t] * y[t][None, :]
    n = jnp.stack([G[i, i] for i in range(8)], axis=0)
    jlt = (jax.lax.broadcasted_iota(jnp.int32, (8, 8, 1), 1)
           < jax.lax.broadcasted_iota(jnp.int32, (8, 8, 1), 0))
    gt = n[None, :, :] > n[:, None, :]
    eq = (n[None, :, :] == n[:, None, :]) & jlt
    rank = jnp.sum((gt | eq).astype(jnp.int32), axis=1)
    P = (rank[None, :, :] == jnp.arange(8, dtype=jnp.int32)[:, None, None]
         ).astype(jnp.float32)
    tmp = jnp.zeros((8, 8, Bb), jnp.float32)
    for i in range(8):
        tmp = tmp + P[:, i][:, None, :] * G[i][None, :, :]
    Gp = jnp.zeros((8, 8, Bb), jnp.float32)
    for j in range(8):
        Gp = Gp + tmp[:, j][:, None, :] * P[:, j][None, :, :]
    zp = jnp.zeros((8, Bb), jnp.float32)
    for i in range(8):
        zp = zp + P[:, i] * z[i][None, :]
    L = [[None] * 8 for _ in range(8)]
    D = [None] * 8
    for j in range(8):
        acc = Gp[j, j]
        for k in range(j):
            acc = acc - L[j][k] * L[j][k] * D[k]
        D[j] = acc
        for i in range(j + 1, 8):
            a2 = Gp[i, j]
            for k in range(j):
                a2 = a2 - L[i][k] * L[j][k] * D[k]
            L[i][j] = a2 / D[j]
    u = [None] * 8
    for i in range(8):
        acc = zp[i]
        for k in range(i):
            acc = acc - L[i][k] * u[k]
        u[i] = acc
    v = [u[i] / D[i] for i in range(8)]
    dists = jnp.zeros((1, Bb), jnp.float32)
    packed = jnp.zeros((1, Bb), jnp.int32)
    for stream in range(_NS):
        si = _NS - 1 - stream
        Pcur = dists.shape[0]
        m = jnp.zeros((Pcur, Bb), jnp.float32)
        for j in range(si + 1, 8):
            ind = (packed >> (2 * j)) & 3
            sym = (2.0 * ind.astype(jnp.float32) - 3.0) * _ISQ10
            m = m + L[j][si][None, :] * sym
        resid = v[si][None, :] - m
        dd = D[si][None, :]
        newd, newp = [], []
        for c in range(_NPAM):
            t = resid - _CONST[c]
            newd.append(dists + dd * t * t)
            newp.append(packed | np.int32(c << (2 * si)))
        if 4 * Pcur <= _K:
            dists = jnp.concatenate(newd, axis=0)
            packed = jnp.concatenate(newp, axis=0)
        else:
            d_e = jnp.stack(newd, axis=1).reshape(4 * Pcur, Bb)
            p_e = jnp.stack(newp, axis=1).reshape(4 * Pcur, Bb)
            dists, packed = _select64(d_e, p_e)
    ind = []
    for j in range(8):
        ind.append((packed >> (2 * rank[j][None, :])) & 3)
    for i in range(4):
        qam = ind[i] * _NPAM + ind[i + 4]
        for bit in range(4):
            b = (qam >> (3 - bit)) & 1
            d0 = jnp.min(jnp.where(b == 0, dists, _BIG), axis=0)
            d1 = jnp.min(jnp.where(b == 1, dists, _BIG), axis=0)
            out_ref[i * 4 + bit, :] = jnp.clip(d0 - d1, -_CLIP, _CLIP)


def kernel(y, h, s_diag):
    B = y.shape[0]
    Bblk = 2048
    ht = jnp.transpose(h, (1, 2, 0))
    yt = jnp.transpose(y, (1, 0))
    st = jnp.transpose(s_diag, (1, 0))
    out = pl.pallas_call(
        _kbest_block,
        grid=(B // Bblk,),
        in_specs=[
            pl.BlockSpec((16, 8, Bblk), lambda i: (0, 0, i)),
            pl.BlockSpec((16, Bblk), lambda i: (0, i)),
            pl.BlockSpec((16, Bblk), lambda i: (0, i)),
        ],
        out_specs=pl.BlockSpec((16, Bblk), lambda i: (0, i)),
        out_shape=jax.ShapeDtypeStruct((16, B), jnp.float32),
    )(ht, yt, st)
    return jnp.transpose(out, (1, 0)).reshape(B, 4, 4)


# speedup vs baseline: 3.2257x; 1.1041x over previous
import numpy as np
import jax
import jax.numpy as jnp
from jax.experimental import pallas as pl

_NS = 8
_NPAM = 4
_K = 64
_CLIP = 20.0
_BIG = 1e9
_ISQ10 = np.float32(1.0 / np.sqrt(10.0))
_CONST = (np.array([-3.0, -1.0, 1.0, 3.0], dtype=np.float32) * _ISQ10)


def _ce(d_arr, p_arr, dist, asc_mask=None):
    N, B = d_arr.shape
    nb = N // (2 * dist)
    d = d_arr.reshape(nb, 2, dist, B)
    p = p_arr.reshape(nb, 2, dist, B)
    a, b = d[:, 0], d[:, 1]
    pa, pb = p[:, 0], p[:, 1]
    le = a <= b
    if asc_mask is None:
        le_eff = le
    else:
        le_eff = le == asc_mask
    first = jnp.where(le_eff, a, b)
    second = jnp.where(le_eff, b, a)
    pfirst = jnp.where(le_eff, pa, pb)
    psecond = jnp.where(le_eff, pb, pa)
    d_out = jnp.stack([first, second], axis=1).reshape(N, B)
    p_out = jnp.stack([pfirst, psecond], axis=1).reshape(N, B)
    return d_out, p_out


def _blk_iota(nb, dist):
    return jax.lax.broadcasted_iota(jnp.int32, (nb, 1, 1), 0) * (2 * dist)


def _select64(d_arr, p_arr):
    N = 256
    for k in range(2, 7):
        d = 2 ** (k - 1)
        while d >= 1:
            nb = N // (2 * d)
            asc = ((_blk_iota(nb, d) >> k) & 1) == 0
            d_arr, p_arr = _ce(d_arr, p_arr, d, asc)
            d //= 2
    d_arr, p_arr = _ce(d_arr, p_arr, 64)
    d2 = jnp.concatenate([d_arr[0:64], d_arr[128:192]], axis=0)
    p2 = jnp.concatenate([p_arr[0:64], p_arr[128:192]], axis=0)
    for d in (32, 16, 8, 4, 2, 1):
        nb = 128 // (2 * d)
        asc = _blk_iota(nb, d) < 64
        d2, p2 = _ce(d2, p2, d, asc)
    d2, p2 = _ce(d2, p2, 64)
    return d2[0:64], p2[0:64]


def _kbest_block(h_ref, y_ref, s_ref, out_ref):
    h = h_ref[...]
    y = y_ref[...]
    s = s_ref[...]
    Bb = y.shape[-1]
    sinv = 1.0 / s
    hd = h * sinv[:, None, :]
    G = jnp.zeros((8, 8, Bb), jnp.float32)
    z = jnp.zeros((8, Bb), jnp.float32)
    for t in range(16):
        G = G + hd[t][:, None, :] * h[t][None, :, :]
        z = z + hd[t] * y[t][None, :]
    n = jnp.stack([G[i, i] for i in range(8)], axis=0)
    jlt = (jax.lax.broadcasted_iota(jnp.int32, (8, 8, 1), 1)
           < jax.lax.broadcasted_iota(jnp.int32, (8, 8, 1), 0))
    gt = n[None, :, :] > n[:, None, :]
    eq = (n[None, :, :] == n[:, None, :]) & jlt
    rank = jnp.sum((gt | eq).astype(jnp.int32), axis=1)
    P = (rank[None, :, :] == jnp.arange(8, dtype=jnp.int32)[:, None, None]
         ).astype(jnp.float32)
    tmp = jnp.zeros((8, 8, Bb), jnp.float32)
    for i in range(8):
        tmp = tmp + P[:, i][:, None, :] * G[i][None, :, :]
    Gp = jnp.zeros((8, 8, Bb), jnp.float32)
    for j in range(8):
        Gp = Gp + tmp[:, j][:, None, :] * P[:, j][None, :, :]
    zp = jnp.zeros((8, Bb), jnp.float32)
    for i in range(8):
        zp = zp + P[:, i] * z[i][None, :]
    L = [[None] * 8 for _ in range(8)]
    D = [None] * 8
    for j in range(8):
        acc = Gp[j, j]
        for k in range(j):
            acc = acc - L[j][k] * L[j][k] * D[k]
        D[j] = acc
        for i in range(j + 1, 8):
            a2 = Gp[i, j]
            for k in range(j):
                a2 = a2 - L[i][k] * L[j][k] * D[k]
            L[i][j] = a2 / D[j]
    u = [None] * 8
    for i in range(8):
        acc = zp[i]
        for k in range(i):
            acc = acc - L[i][k] * u[k]
        u[i] = acc
    v = [u[i] / D[i] for i in range(8)]
    dists = jnp.zeros((1, Bb), jnp.float32)
    packed = jnp.zeros((1, Bb), jnp.int32)
    for stream in range(_NS):
        si = _NS - 1 - stream
        Pcur = dists.shape[0]
        m = jnp.zeros((Pcur, Bb), jnp.float32)
        for j in range(si + 1, 8):
            ind = (packed >> (2 * j)) & 3
            sym = (2.0 * ind.astype(jnp.float32) - 3.0) * _ISQ10
            m = m + L[j][si][None, :] * sym
        resid = v[si][None, :] - m
        dd = D[si][None, :]
        newd, newp = [], []
        for c in range(_NPAM):
            t = resid - _CONST[c]
            newd.append(dists + dd * t * t)
            newp.append(packed | np.int32(c << (2 * si)))
        if 4 * Pcur <= _K:
            dists = jnp.concatenate(newd, axis=0)
            packed = jnp.concatenate(newp, axis=0)
        else:
            d_e = jnp.stack(newd, axis=1).reshape(4 * Pcur, Bb)
            p_e = jnp.stack(newp, axis=1).reshape(4 * Pcur, Bb)
            dists, packed = _select64(d_e, p_e)
    ind = []
    for j in range(8):
        ind.append((packed >> (2 * rank[j][None, :])) & 3)
    for i in range(4):
        qam = ind[i] * _NPAM + ind[i + 4]
        for bit in range(4):
            b = (qam >> (3 - bit)) & 1
            d0 = jnp.min(jnp.where(b == 0, dists, _BIG), axis=0)
            d1 = jnp.min(jnp.where(b == 1, dists, _BIG), axis=0)
            out_ref[i * 4 + bit, :] = jnp.clip(d0 - d1, -_CLIP, _CLIP)


def kernel(y, h, s_diag):
    B = y.shape[0]
    Bblk = 4096
    ht = jnp.transpose(h, (1, 2, 0))
    yt = jnp.transpose(y, (1, 0))
    st = jnp.transpose(s_diag, (1, 0))
    out = pl.pallas_call(
        _kbest_block,
        grid=(B // Bblk,),
        in_specs=[
            pl.BlockSpec((16, 8, Bblk), lambda i: (0, 0, i)),
            pl.BlockSpec((16, Bblk), lambda i: (0, i)),
            pl.BlockSpec((16, Bblk), lambda i: (0, i)),
        ],
        out_specs=pl.BlockSpec((16, Bblk), lambda i: (0, i)),
        out_shape=jax.ShapeDtypeStruct((16, B), jnp.float32),
    )(ht, yt, st)
    return jnp.transpose(out, (1, 0)).reshape(B, 4, 4)
